# Initial kernel scaffold; baseline (speedup 1.0000x reference)
import jax
import jax.numpy as jnp
from jax.experimental import pallas as pl


def kernel(graph, n_feat, W_rel_in, b_in, W_root_in, W_rel_int, b_int, W_root_int, W_rel_out, b_out, W_root_out, W_lin, b_lin):
    raise NotImplementedError("write your pallas kernel here")


# speedup vs baseline: 7.3924x; 7.3924x over previous
import functools

import jax
import jax.numpy as jnp
from jax import lax
from jax.experimental import pallas as pl
from jax.experimental.pallas import tpu as pltpu
from jax.experimental.pallas import tpu_sc as plsc

N_REAL = 10000
NP = 10240
FP = 32
NC, NS = 2, 16
NW = NC * NS
E_REAL = 320000
CHUNK = 128
EPW = 10240
NCHUNK = EPW // CHUNK
EP = EPW * NW
RPT = NP // NS
BN = 1024
GRID = NP // BN


def _sc_agg(p, srcp, dstp, zeros):
    mesh = plsc.VectorSubcoreMesh(core_axis_name="c", subcore_axis_name="s")

    @functools.partial(
        pl.kernel,
        out_type=jax.ShapeDtypeStruct((NC * NP, FP), jnp.float32),
        mesh=mesh,
        compiler_params=pltpu.CompilerParams(use_tc_tiling_on_sc=False),
        scratch_types=[
            pltpu.VMEM((NCHUNK, CHUNK), jnp.int32),
            pltpu.VMEM((NCHUNK, CHUNK), jnp.int32),
            pltpu.VMEM((CHUNK, FP), jnp.float32),
            pltpu.VMEM((RPT, FP), jnp.float32),
            pltpu.SemaphoreType.DMA,
            pltpu.VMEM_SHARED((NP, FP), jnp.float32),
        ],
    )
    def k(p_hbm, src_hbm, dst_hbm, z_hbm, agg_hbm,
          src_v, dst_v, msg_v, buf_v, sem, acc_sh):
        cid = lax.axis_index("c")
        sid = lax.axis_index("s")
        wid = sid * NC + cid
        r0 = sid * RPT
        pltpu.sync_copy(z_hbm.at[pl.ds(r0, RPT)], acc_sh.at[pl.ds(r0, RPT)])
        plsc.subcore_barrier()
        pltpu.sync_copy(src_hbm.at[wid], src_v)
        pltpu.sync_copy(dst_hbm.at[wid], dst_v)

        def body(j, carry):
            pltpu.async_copy(p_hbm.at[src_v.at[j]], msg_v, sem).wait()
            pltpu.sync_copy(msg_v, acc_sh.at[dst_v.at[j]], add=True)
            return carry

        lax.fori_loop(0, NCHUNK, body, 0)
        plsc.subcore_barrier()
        pltpu.sync_copy(acc_sh.at[pl.ds(r0, RPT)], buf_v)
        pltpu.sync_copy(buf_v, agg_hbm.at[pl.ds(cid * NP + r0, RPT)])

    return k(p, srcp, dstp, zeros)


def _matmul_body(x_ref, w_ref, o_ref):
    o_ref[...] = jnp.dot(x_ref[...], w_ref[...],
                         preferred_element_type=jnp.float32)


def _project(x, w):
    d_in, d_out = w.shape
    return pl.pallas_call(
        _matmul_body,
        grid=(GRID,),
        in_specs=[
            pl.BlockSpec((BN, d_in), lambda i: (i, 0)),
            pl.BlockSpec((d_in, d_out), lambda i: (0, 0)),
        ],
        out_specs=pl.BlockSpec((BN, d_out), lambda i: (i, 0)),
        out_shape=jax.ShapeDtypeStruct((NP, d_out), jnp.float32),
    )(x, w)


def _dense_body(a0_ref, a1_ref, h_ref, wr_ref, b_ref, wn_ref, hn_ref, pn_ref):
    i = pl.program_id(0)
    agg = a0_ref[...] + a1_ref[...]
    pre = agg + b_ref[...] + jnp.dot(h_ref[...], wr_ref[...],
                                     preferred_element_type=jnp.float32)
    hv = jnp.maximum(pre, 0.0)
    rows = i * BN + lax.broadcasted_iota(jnp.int32, (BN, FP), 0)
    hv = jnp.where(rows < N_REAL, hv, 0.0)
    hn_ref[...] = hv
    pn_ref[...] = jnp.dot(hv, wn_ref[...], preferred_element_type=jnp.float32)


def _dense(a0, a1, h, w_root, b, w_next):
    d_in = h.shape[1]
    return pl.pallas_call(
        _dense_body,
        grid=(GRID,),
        in_specs=[
            pl.BlockSpec((BN, FP), lambda i: (i, 0)),
            pl.BlockSpec((BN, FP), lambda i: (i, 0)),
            pl.BlockSpec((BN, d_in), lambda i: (i, 0)),
            pl.BlockSpec((d_in, FP), lambda i: (0, 0)),
            pl.BlockSpec((1, FP), lambda i: (0, 0)),
            pl.BlockSpec((FP, FP), lambda i: (0, 0)),
        ],
        out_specs=[
            pl.BlockSpec((BN, FP), lambda i: (i, 0)),
            pl.BlockSpec((BN, FP), lambda i: (i, 0)),
        ],
        out_shape=[jax.ShapeDtypeStruct((NP, FP), jnp.float32)] * 2,
    )(a0, a1, h, w_root, b, w_next)


def _final_body(a0_ref, a1_ref, h_ref, wre_ref, wro_ref, bo_ref,
                wlin_ref, blin_ref, out_ref, acc_ref):
    i = pl.program_id(0)
    agg = a0_ref[...] + a1_ref[...]
    pre = (jnp.dot(agg, wre_ref[...], preferred_element_type=jnp.float32)
           + bo_ref[...]
           + jnp.dot(h_ref[...], wro_ref[...],
                     preferred_element_type=jnp.float32))
    hv = jnp.maximum(pre, 0.0)
    rows = i * BN + lax.broadcasted_iota(jnp.int32, (BN, 128), 0)
    hv = jnp.where(rows < N_REAL, hv, 0.0)
    s = jnp.sum(hv, axis=0, keepdims=True)

    @pl.when(i == 0)
    def _():
        acc_ref[...] = s

    @pl.when(i > 0)
    def _():
        acc_ref[...] = acc_ref[...] + s

    @pl.when(i == GRID - 1)
    def _():
        v = acc_ref[...]
        for t in range(4):
            v = jnp.maximum(
                jnp.dot(v, wlin_ref[t], preferred_element_type=jnp.float32)
                + blin_ref[t][None], 0.0)
        out_ref[...] = v


def _final(a0, a1, h, w_re, w_ro, b_o, w_lin, b_lin):
    return pl.pallas_call(
        _final_body,
        grid=(GRID,),
        in_specs=[
            pl.BlockSpec((BN, FP), lambda i: (i, 0)),
            pl.BlockSpec((BN, FP), lambda i: (i, 0)),
            pl.BlockSpec((BN, FP), lambda i: (i, 0)),
            pl.BlockSpec((FP, 128), lambda i: (0, 0)),
            pl.BlockSpec((FP, 128), lambda i: (0, 0)),
            pl.BlockSpec((1, 128), lambda i: (0, 0)),
            pl.BlockSpec((4, 128, 128), lambda i: (0, 0, 0)),
            pl.BlockSpec((4, 128), lambda i: (0, 0)),
        ],
        out_specs=pl.BlockSpec((1, 128), lambda i: (0, 0)),
        out_shape=jax.ShapeDtypeStruct((1, 128), jnp.float32),
        scratch_shapes=[pltpu.VMEM((1, 128), jnp.float32)],
    )(a0, a1, h, w_re, w_ro, b_o, w_lin, b_lin)


def _padw(w, r, c):
    out = jnp.zeros((r, c), jnp.float32)
    return out.at[:w.shape[0], :w.shape[1]].set(w)


def kernel(graph, n_feat, W_rel_in, b_in, W_root_in, W_rel_int, b_int,
           W_root_int, W_rel_out, b_out, W_root_out, W_lin, b_lin):
    f32 = jnp.float32
    x = jnp.zeros((NP, 128), f32).at[:N_REAL].set(n_feat)

    pad_e = EP - E_REAL
    pad_idx = jnp.full((pad_e,), NP - 1, jnp.int32)
    srcp = jnp.concatenate([graph[0], pad_idx]).reshape(NW, NCHUNK, CHUNK)
    dstp = jnp.concatenate([graph[1], pad_idx]).reshape(NW, NCHUNK, CHUNK)
    zeros = jnp.zeros((NP, FP), f32)

    w_rel_in = _padw(W_rel_in, 128, FP)
    w_root_in = _padw(W_root_in, 128, FP)
    b0 = _padw(b_in[None], 1, FP)
    w_rel_mid = [_padw(W_rel_int[i], FP, FP) for i in range(W_rel_int.shape[0])]
    w_root_mid = [_padw(W_root_int[i], FP, FP) for i in range(W_root_int.shape[0])]
    b_mid = [_padw(b_int[i][None], 1, FP) for i in range(b_int.shape[0])]
    w_re_out = _padw(W_rel_out, FP, 128)
    w_ro_out = _padw(W_root_out, FP, 128)
    eye = jnp.eye(FP, dtype=f32)

    w_roots = [w_root_in] + w_root_mid
    bs = [b0] + b_mid
    w_nexts = w_rel_mid + [eye]

    p = _project(x, w_rel_in)
    h = x
    for l in range(len(w_roots)):
        agg = _sc_agg(p, srcp, dstp, zeros)
        h, p = _dense(agg[:NP], agg[NP:], h, w_roots[l], bs[l], w_nexts[l])

    agg = _sc_agg(p, srcp, dstp, zeros)
    v = _final(agg[:NP], agg[NP:], h, w_re_out, w_ro_out, b_out[None],
               W_lin, b_lin)
    return v[0]


# speedup vs baseline: 9.2277x; 1.2483x over previous
import functools

import jax
import jax.numpy as jnp
from jax import lax
from jax.experimental import pallas as pl
from jax.experimental.pallas import tpu as pltpu
from jax.experimental.pallas import tpu_sc as plsc

N_REAL = 10000
NP = 10240
FP = 32
NC, NS = 2, 16
NW = NC * NS
E_REAL = 320000
CHUNK = 128
EPW = 10240
NCHUNK = EPW // CHUNK
EP = EPW * NW
RPT = NP // NS
NBUF = 8
BN = 1024
GRID = NP // BN


def _sc_agg(p, srcp, dstp, zeros):
    mesh = plsc.VectorSubcoreMesh(core_axis_name="c", subcore_axis_name="s")

    @functools.partial(
        pl.kernel,
        out_type=jax.ShapeDtypeStruct((NC * NP, FP), jnp.float32),
        mesh=mesh,
        compiler_params=pltpu.CompilerParams(use_tc_tiling_on_sc=False),
        scratch_types=[
            pltpu.VMEM((NCHUNK, CHUNK), jnp.int32),
            pltpu.VMEM((NCHUNK, CHUNK), jnp.int32),
            pltpu.VMEM((NBUF, CHUNK, FP), jnp.float32),
            pltpu.VMEM((RPT, FP), jnp.float32),
            pltpu.SemaphoreType.DMA((NBUF,)),
            pltpu.SemaphoreType.DMA((NBUF,)),
            pltpu.VMEM_SHARED((NP, FP), jnp.float32),
        ],
    )
    def k(p_hbm, src_hbm, dst_hbm, z_hbm, agg_hbm,
          src_v, dst_v, msg_v, buf_v, semg, sems, acc_sh):
        cid = lax.axis_index("c")
        sid = lax.axis_index("s")
        wid = sid * NC + cid
        r0 = sid * RPT
        pltpu.sync_copy(z_hbm.at[pl.ds(r0, RPT)], acc_sh.at[pl.ds(r0, RPT)])
        plsc.subcore_barrier()
        pltpu.sync_copy(src_hbm.at[wid], src_v)
        pltpu.sync_copy(dst_hbm.at[wid], dst_v)

        def fire_g(j, b):
            pltpu.make_async_copy(
                p_hbm.at[src_v.at[j]], msg_v.at[b], semg.at[b]).start()

        def wait_g(b):
            pltpu.make_async_copy(
                p_hbm.at[src_v.at[0]], msg_v.at[b], semg.at[b]).wait()

        def fire_s(j, b):
            pltpu.make_async_copy(
                msg_v.at[b], acc_sh.at[dst_v.at[j]],
                sems.at[b]).start(add=True)

        def wait_s(b):
            pltpu.make_async_copy(
                msg_v.at[b], acc_sh.at[dst_v.at[0]], sems.at[b]).wait()

        half = NBUF // 2
        for b in range(half):
            fire_g(b, b)

        def body(i, carry):
            j0 = i * NBUF
            for b in range(half):
                wait_g(b)
                fire_s(j0 + b, b)
            for b in range(half, NBUF):
                @pl.when(i > 0)
                def _():
                    wait_s(b)
                fire_g(j0 + b, b)
            for b in range(half, NBUF):
                wait_g(b)
                fire_s(j0 + b, b)
            for b in range(half):
                wait_s(b)
                jn = j0 + NBUF + b
                @pl.when(jn < NCHUNK)
                def _():
                    fire_g(jn, b)
            return carry

        lax.fori_loop(0, NCHUNK // NBUF, body, 0)
        for b in range(half, NBUF):
            wait_s(b)
        plsc.subcore_barrier()
        pltpu.sync_copy(acc_sh.at[pl.ds(r0, RPT)], buf_v)
        pltpu.sync_copy(buf_v, agg_hbm.at[pl.ds(cid * NP + r0, RPT)])

    return k(p, srcp, dstp, zeros)


def _matmul_body(x_ref, w_ref, o_ref):
    o_ref[...] = jnp.dot(x_ref[...], w_ref[...],
                         preferred_element_type=jnp.float32)


def _project(x, w):
    d_in, d_out = w.shape
    return pl.pallas_call(
        _matmul_body,
        grid=(GRID,),
        in_specs=[
            pl.BlockSpec((BN, d_in), lambda i: (i, 0)),
            pl.BlockSpec((d_in, d_out), lambda i: (0, 0)),
        ],
        out_specs=pl.BlockSpec((BN, d_out), lambda i: (i, 0)),
        out_shape=jax.ShapeDtypeStruct((NP, d_out), jnp.float32),
    )(x, w)


def _dense_body(a0_ref, a1_ref, h_ref, wr_ref, b_ref, wn_ref, hn_ref, pn_ref):
    i = pl.program_id(0)
    agg = a0_ref[...] + a1_ref[...]
    pre = agg + b_ref[...] + jnp.dot(h_ref[...], wr_ref[...],
                                     preferred_element_type=jnp.float32)
    hv = jnp.maximum(pre, 0.0)
    rows = i * BN + lax.broadcasted_iota(jnp.int32, (BN, FP), 0)
    hv = jnp.where(rows < N_REAL, hv, 0.0)
    hn_ref[...] = hv
    pn_ref[...] = jnp.dot(hv, wn_ref[...], preferred_element_type=jnp.float32)


def _dense(a0, a1, h, w_root, b, w_next):
    d_in = h.shape[1]
    return pl.pallas_call(
        _dense_body,
        grid=(GRID,),
        in_specs=[
            pl.BlockSpec((BN, FP), lambda i: (i, 0)),
            pl.BlockSpec((BN, FP), lambda i: (i, 0)),
            pl.BlockSpec((BN, d_in), lambda i: (i, 0)),
            pl.BlockSpec((d_in, FP), lambda i: (0, 0)),
            pl.BlockSpec((1, FP), lambda i: (0, 0)),
            pl.BlockSpec((FP, FP), lambda i: (0, 0)),
        ],
        out_specs=[
            pl.BlockSpec((BN, FP), lambda i: (i, 0)),
            pl.BlockSpec((BN, FP), lambda i: (i, 0)),
        ],
        out_shape=[jax.ShapeDtypeStruct((NP, FP), jnp.float32)] * 2,
    )(a0, a1, h, w_root, b, w_next)


def _final_body(a0_ref, a1_ref, h_ref, wre_ref, wro_ref, bo_ref,
                wlin_ref, blin_ref, out_ref, acc_ref):
    i = pl.program_id(0)
    agg = a0_ref[...] + a1_ref[...]
    pre = (jnp.dot(agg, wre_ref[...], preferred_element_type=jnp.float32)
           + bo_ref[...]
           + jnp.dot(h_ref[...], wro_ref[...],
                     preferred_element_type=jnp.float32))
    hv = jnp.maximum(pre, 0.0)
    rows = i * BN + lax.broadcasted_iota(jnp.int32, (BN, 128), 0)
    hv = jnp.where(rows < N_REAL, hv, 0.0)
    s = jnp.sum(hv, axis=0, keepdims=True)

    @pl.when(i == 0)
    def _():
        acc_ref[...] = s

    @pl.when(i > 0)
    def _():
        acc_ref[...] = acc_ref[...] + s

    @pl.when(i == GRID - 1)
    def _():
        v = acc_ref[...]
        for t in range(4):
            v = jnp.maximum(
                jnp.dot(v, wlin_ref[t], preferred_element_type=jnp.float32)
                + blin_ref[t][None], 0.0)
        out_ref[...] = v


def _final(a0, a1, h, w_re, w_ro, b_o, w_lin, b_lin):
    return pl.pallas_call(
        _final_body,
        grid=(GRID,),
        in_specs=[
            pl.BlockSpec((BN, FP), lambda i: (i, 0)),
            pl.BlockSpec((BN, FP), lambda i: (i, 0)),
            pl.BlockSpec((BN, FP), lambda i: (i, 0)),
            pl.BlockSpec((FP, 128), lambda i: (0, 0)),
            pl.BlockSpec((FP, 128), lambda i: (0, 0)),
            pl.BlockSpec((1, 128), lambda i: (0, 0)),
            pl.BlockSpec((4, 128, 128), lambda i: (0, 0, 0)),
            pl.BlockSpec((4, 128), lambda i: (0, 0)),
        ],
        out_specs=pl.BlockSpec((1, 128), lambda i: (0, 0)),
        out_shape=jax.ShapeDtypeStruct((1, 128), jnp.float32),
        scratch_shapes=[pltpu.VMEM((1, 128), jnp.float32)],
    )(a0, a1, h, w_re, w_ro, b_o, w_lin, b_lin)


def _padw(w, r, c):
    out = jnp.zeros((r, c), jnp.float32)
    return out.at[:w.shape[0], :w.shape[1]].set(w)


def kernel(graph, n_feat, W_rel_in, b_in, W_root_in, W_rel_int, b_int,
           W_root_int, W_rel_out, b_out, W_root_out, W_lin, b_lin):
    f32 = jnp.float32
    x = jnp.zeros((NP, 128), f32).at[:N_REAL].set(n_feat)

    pad_e = EP - E_REAL
    pad_idx = jnp.full((pad_e,), NP - 1, jnp.int32)
    srcp = jnp.concatenate([graph[0], pad_idx]).reshape(NW, NCHUNK, CHUNK)
    dstp = jnp.concatenate([graph[1], pad_idx]).reshape(NW, NCHUNK, CHUNK)
    zeros = jnp.zeros((NP, FP), f32)

    w_rel_in = _padw(W_rel_in, 128, FP)
    w_root_in = _padw(W_root_in, 128, FP)
    b0 = _padw(b_in[None], 1, FP)
    w_rel_mid = [_padw(W_rel_int[i], FP, FP) for i in range(W_rel_int.shape[0])]
    w_root_mid = [_padw(W_root_int[i], FP, FP) for i in range(W_root_int.shape[0])]
    b_mid = [_padw(b_int[i][None], 1, FP) for i in range(b_int.shape[0])]
    w_re_out = _padw(W_rel_out, FP, 128)
    w_ro_out = _padw(W_root_out, FP, 128)
    eye = jnp.eye(FP, dtype=f32)

    w_roots = [w_root_in] + w_root_mid
    bs = [b0] + b_mid
    w_nexts = w_rel_mid + [eye]

    p = _project(x, w_rel_in)
    h = x
    for l in range(len(w_roots)):
        agg = _sc_agg(p, srcp, dstp, zeros)
        h, p = _dense(agg[:NP], agg[NP:], h, w_roots[l], bs[l], w_nexts[l])

    agg = _sc_agg(p, srcp, dstp, zeros)
    v = _final(agg[:NP], agg[NP:], h, w_re_out, w_ro_out, b_out[None],
               W_lin, b_lin)
    return v[0]


# speedup vs baseline: 22.9744x; 2.4897x over previous
import functools

import jax
import jax.numpy as jnp
from jax import lax
from jax.experimental import pallas as pl
from jax.experimental.pallas import tpu as pltpu
from jax.experimental.pallas import tpu_sc as plsc

N_REAL = 10000
NP = 10240
FP = 32
NC, NS = 2, 16
NW = NC * NS
E_REAL = 320000
CHUNK = 640
EPW = 10240
NOPS = EPW // CHUNK
EP = EPW * NW
RPT = NP // NS
NBUF = 4
BN = 1024
GRID = NP // BN


def _sc_agg(p, srcp, dstp, zeros):
    mesh = plsc.VectorSubcoreMesh(core_axis_name="c", subcore_axis_name="s")

    @functools.partial(
        pl.kernel,
        out_type=jax.ShapeDtypeStruct((NC * NP, FP), jnp.bfloat16),
        mesh=mesh,
        compiler_params=pltpu.CompilerParams(use_tc_tiling_on_sc=False),
        scratch_types=[
            pltpu.VMEM((NOPS, CHUNK), jnp.int32),
            pltpu.VMEM((NOPS, CHUNK), jnp.int32),
            pltpu.VMEM((NBUF, CHUNK, FP), jnp.bfloat16),
            pltpu.VMEM((RPT, FP), jnp.bfloat16),
            pltpu.SemaphoreType.DMA((NBUF,)),
            pltpu.SemaphoreType.DMA((NBUF,)),
            pltpu.VMEM_SHARED((NP, FP), jnp.bfloat16),
            pltpu.VMEM_SHARED((NP, FP), jnp.bfloat16),
        ],
    )
    def k(p_hbm, src_hbm, dst_hbm, z_hbm, agg_hbm,
          src_v, dst_v, msg_v, buf_v, semg, sems, acc_sh, p_sh):
        cid = lax.axis_index("c")
        sid = lax.axis_index("s")
        wid = sid * NC + cid
        r0 = sid * RPT
        pltpu.sync_copy(z_hbm.at[pl.ds(r0, RPT)], acc_sh.at[pl.ds(r0, RPT)])
        pltpu.sync_copy(p_hbm.at[pl.ds(r0, RPT)], p_sh.at[pl.ds(r0, RPT)])
        plsc.subcore_barrier()
        pltpu.sync_copy(src_hbm.at[wid], src_v)
        pltpu.sync_copy(dst_hbm.at[wid], dst_v)

        def fire_g(j, b):
            pltpu.make_async_copy(
                p_sh.at[src_v.at[j]], msg_v.at[b], semg.at[b]).start()

        def wait_g(b):
            pltpu.make_async_copy(
                p_sh.at[src_v.at[0]], msg_v.at[b], semg.at[b]).wait()

        def fire_s(j, b):
            pltpu.make_async_copy(
                msg_v.at[b], acc_sh.at[dst_v.at[j]],
                sems.at[b]).start(add=True)

        def wait_s(b):
            pltpu.make_async_copy(
                msg_v.at[b], acc_sh.at[dst_v.at[0]], sems.at[b]).wait()

        half = NBUF // 2
        for b in range(half):
            fire_g(b, b)

        def body(i, carry):
            j0 = i * NBUF
            for b in range(half):
                wait_g(b)
                fire_s(j0 + b, b)
            for b in range(half, NBUF):
                @pl.when(i > 0)
                def _():
                    wait_s(b)
                fire_g(j0 + b, b)
            for b in range(half, NBUF):
                wait_g(b)
                fire_s(j0 + b, b)
            for b in range(half):
                wait_s(b)
                jn = j0 + NBUF + b
                @pl.when(jn < NOPS)
                def _():
                    fire_g(jn, b)
            return carry

        lax.fori_loop(0, NOPS // NBUF, body, 0)
        for b in range(half, NBUF):
            wait_s(b)
        plsc.subcore_barrier()
        pltpu.sync_copy(acc_sh.at[pl.ds(r0, RPT)], buf_v)
        pltpu.sync_copy(buf_v, agg_hbm.at[pl.ds(cid * NP + r0, RPT)])

    return k(p, srcp, dstp, zeros)


def _matmul_body(x_ref, w_ref, o_ref):
    o_ref[...] = jnp.dot(x_ref[...], w_ref[...],
                         preferred_element_type=jnp.float32
                         ).astype(jnp.bfloat16)


def _project(x, w):
    d_in, d_out = w.shape
    return pl.pallas_call(
        _matmul_body,
        grid=(GRID,),
        in_specs=[
            pl.BlockSpec((BN, d_in), lambda i: (i, 0)),
            pl.BlockSpec((d_in, d_out), lambda i: (0, 0)),
        ],
        out_specs=pl.BlockSpec((BN, d_out), lambda i: (i, 0)),
        out_shape=jax.ShapeDtypeStruct((NP, d_out), jnp.bfloat16),
    )(x, w)


def _dense_body(a0_ref, a1_ref, h_ref, wr_ref, b_ref, wn_ref, hn_ref, pn_ref):
    i = pl.program_id(0)
    agg = a0_ref[...].astype(jnp.float32) + a1_ref[...].astype(jnp.float32)
    pre = agg + b_ref[...] + jnp.dot(h_ref[...], wr_ref[...],
                                     preferred_element_type=jnp.float32)
    hv = jnp.maximum(pre, 0.0)
    rows = i * BN + lax.broadcasted_iota(jnp.int32, (BN, FP), 0)
    hv = jnp.where(rows < N_REAL, hv, 0.0)
    hn_ref[...] = hv
    pn_ref[...] = jnp.dot(hv, wn_ref[...], preferred_element_type=jnp.float32
                          ).astype(jnp.bfloat16)


def _dense(a0, a1, h, w_root, b, w_next):
    d_in = h.shape[1]
    return pl.pallas_call(
        _dense_body,
        grid=(GRID,),
        in_specs=[
            pl.BlockSpec((BN, FP), lambda i: (i, 0)),
            pl.BlockSpec((BN, FP), lambda i: (i, 0)),
            pl.BlockSpec((BN, d_in), lambda i: (i, 0)),
            pl.BlockSpec((d_in, FP), lambda i: (0, 0)),
            pl.BlockSpec((1, FP), lambda i: (0, 0)),
            pl.BlockSpec((FP, FP), lambda i: (0, 0)),
        ],
        out_specs=[
            pl.BlockSpec((BN, FP), lambda i: (i, 0)),
            pl.BlockSpec((BN, FP), lambda i: (i, 0)),
        ],
        out_shape=[jax.ShapeDtypeStruct((NP, FP), jnp.float32),
                   jax.ShapeDtypeStruct((NP, FP), jnp.bfloat16)],
    )(a0, a1, h, w_root, b, w_next)


def _final_body(a0_ref, a1_ref, h_ref, wre_ref, wro_ref, bo_ref,
                wlin_ref, blin_ref, out_ref, acc_ref):
    i = pl.program_id(0)
    agg = a0_ref[...].astype(jnp.float32) + a1_ref[...].astype(jnp.float32)
    pre = (jnp.dot(agg, wre_ref[...], preferred_element_type=jnp.float32)
           + bo_ref[...]
           + jnp.dot(h_ref[...], wro_ref[...],
                     preferred_element_type=jnp.float32))
    hv = jnp.maximum(pre, 0.0)
    rows = i * BN + lax.broadcasted_iota(jnp.int32, (BN, 128), 0)
    hv = jnp.where(rows < N_REAL, hv, 0.0)
    s = jnp.sum(hv, axis=0, keepdims=True)

    @pl.when(i == 0)
    def _():
        acc_ref[...] = s

    @pl.when(i > 0)
    def _():
        acc_ref[...] = acc_ref[...] + s

    @pl.when(i == GRID - 1)
    def _():
        v = acc_ref[...]
        for t in range(4):
            v = jnp.maximum(
                jnp.dot(v, wlin_ref[t], preferred_element_type=jnp.float32)
                + blin_ref[t][None], 0.0)
        out_ref[...] = v


def _final(a0, a1, h, w_re, w_ro, b_o, w_lin, b_lin):
    return pl.pallas_call(
        _final_body,
        grid=(GRID,),
        in_specs=[
            pl.BlockSpec((BN, FP), lambda i: (i, 0)),
            pl.BlockSpec((BN, FP), lambda i: (i, 0)),
            pl.BlockSpec((BN, FP), lambda i: (i, 0)),
            pl.BlockSpec((FP, 128), lambda i: (0, 0)),
            pl.BlockSpec((FP, 128), lambda i: (0, 0)),
            pl.BlockSpec((1, 128), lambda i: (0, 0)),
            pl.BlockSpec((4, 128, 128), lambda i: (0, 0, 0)),
            pl.BlockSpec((4, 128), lambda i: (0, 0)),
        ],
        out_specs=pl.BlockSpec((1, 128), lambda i: (0, 0)),
        out_shape=jax.ShapeDtypeStruct((1, 128), jnp.float32),
        scratch_shapes=[pltpu.VMEM((1, 128), jnp.float32)],
    )(a0, a1, h, w_re, w_ro, b_o, w_lin, b_lin)


def _padw(w, r, c):
    out = jnp.zeros((r, c), jnp.float32)
    return out.at[:w.shape[0], :w.shape[1]].set(w)


def kernel(graph, n_feat, W_rel_in, b_in, W_root_in, W_rel_int, b_int,
           W_root_int, W_rel_out, b_out, W_root_out, W_lin, b_lin):
    f32 = jnp.float32
    x = jnp.zeros((NP, 128), f32).at[:N_REAL].set(n_feat)

    pad_e = EP - E_REAL
    pad_idx = jnp.full((pad_e,), NP - 1, jnp.int32)
    srcp = jnp.concatenate([graph[0], pad_idx]).reshape(NW, NOPS, CHUNK)
    dstp = jnp.concatenate([graph[1], pad_idx]).reshape(NW, NOPS, CHUNK)
    zeros = jnp.zeros((NP, FP), jnp.bfloat16)

    w_rel_in = _padw(W_rel_in, 128, FP)
    w_root_in = _padw(W_root_in, 128, FP)
    b0 = _padw(b_in[None], 1, FP)
    w_rel_mid = [_padw(W_rel_int[i], FP, FP) for i in range(W_rel_int.shape[0])]
    w_root_mid = [_padw(W_root_int[i], FP, FP) for i in range(W_root_int.shape[0])]
    b_mid = [_padw(b_int[i][None], 1, FP) for i in range(b_int.shape[0])]
    w_re_out = _padw(W_rel_out, FP, 128)
    w_ro_out = _padw(W_root_out, FP, 128)
    eye = jnp.eye(FP, dtype=f32)

    w_roots = [w_root_in] + w_root_mid
    bs = [b0] + b_mid
    w_nexts = w_rel_mid + [eye]

    p = _project(x, w_rel_in)
    h = x
    for l in range(len(w_roots)):
        agg = _sc_agg(p, srcp, dstp, zeros)
        h, p = _dense(agg[:NP], agg[NP:], h, w_roots[l], bs[l], w_nexts[l])

    agg = _sc_agg(p, srcp, dstp, zeros)
    v = _final(agg[:NP], agg[NP:], h, w_re_out, w_ro_out, b_out[None],
               W_lin, b_lin)
    return v[0]


# speedup vs baseline: 23.8976x; 1.0402x over previous
import functools

import jax
import jax.numpy as jnp
from jax import lax
from jax.experimental import pallas as pl
from jax.experimental.pallas import tpu as pltpu
from jax.experimental.pallas import tpu_sc as plsc

N_REAL = 10000
NP = 10240
FP = 32
NC, NS = 2, 16
NW = NC * NS
E_REAL = 320000
CHUNK = 640
EPW = 10240
NOPS = EPW // CHUNK
EP = EPW * NW
RPT = NP // NS
NBUF = 4
BN = 1024
GRID = NP // BN


def _sc_agg(p, srcp, dstp, zeros):
    mesh = plsc.VectorSubcoreMesh(core_axis_name="c", subcore_axis_name="s")

    @functools.partial(
        pl.kernel,
        out_type=jax.ShapeDtypeStruct((NC * NP, FP), jnp.bfloat16),
        mesh=mesh,
        compiler_params=pltpu.CompilerParams(use_tc_tiling_on_sc=False),
        scratch_types=[
            pltpu.VMEM((NOPS, CHUNK), jnp.int32),
            pltpu.VMEM((NOPS, CHUNK), jnp.int32),
            pltpu.VMEM((NBUF, CHUNK, FP), jnp.bfloat16),
            pltpu.VMEM((RPT, FP), jnp.bfloat16),
            pltpu.SemaphoreType.DMA((NBUF,)),
            pltpu.SemaphoreType.DMA((NBUF,)),
            pltpu.SemaphoreType.DMA((4,)),
            pltpu.VMEM_SHARED((NP, FP), jnp.bfloat16),
            pltpu.VMEM_SHARED((NP, FP), jnp.bfloat16),
        ],
    )
    def k(p_hbm, src_hbm, dst_hbm, z_hbm, agg_hbm,
          src_v, dst_v, msg_v, buf_v, semg, sems, semp, acc_sh, p_sh):
        cid = lax.axis_index("c")
        sid = lax.axis_index("s")
        wid = sid * NC + cid
        r0 = sid * RPT
        pro = [
            pltpu.make_async_copy(z_hbm.at[pl.ds(r0, RPT)],
                                  acc_sh.at[pl.ds(r0, RPT)], semp.at[0]),
            pltpu.make_async_copy(p_hbm.at[pl.ds(r0, RPT)],
                                  p_sh.at[pl.ds(r0, RPT)], semp.at[1]),
            pltpu.make_async_copy(src_hbm.at[wid], src_v, semp.at[2]),
            pltpu.make_async_copy(dst_hbm.at[wid], dst_v, semp.at[3]),
        ]
        for c in pro:
            c.start()
        for c in pro:
            c.wait()
        plsc.subcore_barrier()

        def fire_g(j, b):
            pltpu.make_async_copy(
                p_sh.at[src_v.at[j]], msg_v.at[b], semg.at[b]).start()

        def wait_g(b):
            pltpu.make_async_copy(
                p_sh.at[src_v.at[0]], msg_v.at[b], semg.at[b]).wait()

        def fire_s(j, b):
            pltpu.make_async_copy(
                msg_v.at[b], acc_sh.at[dst_v.at[j]],
                sems.at[b]).start(add=True)

        def wait_s(b):
            pltpu.make_async_copy(
                msg_v.at[b], acc_sh.at[dst_v.at[0]], sems.at[b]).wait()

        half = NBUF // 2
        for b in range(half):
            fire_g(b, b)

        def body(i, carry):
            j0 = i * NBUF
            for b in range(half):
                wait_g(b)
                fire_s(j0 + b, b)
            for b in range(half, NBUF):
                @pl.when(i > 0)
                def _():
                    wait_s(b)
                fire_g(j0 + b, b)
            for b in range(half, NBUF):
                wait_g(b)
                fire_s(j0 + b, b)
            for b in range(half):
                wait_s(b)
                jn = j0 + NBUF + b
                @pl.when(jn < NOPS)
                def _():
                    fire_g(jn, b)
            return carry

        lax.fori_loop(0, NOPS // NBUF, body, 0)
        for b in range(half, NBUF):
            wait_s(b)
        plsc.subcore_barrier()
        pltpu.sync_copy(acc_sh.at[pl.ds(r0, RPT)], buf_v)
        pltpu.sync_copy(buf_v, agg_hbm.at[pl.ds(cid * NP + r0, RPT)])

    return k(p, srcp, dstp, zeros)


def _matmul_body(x_ref, w_ref, o_ref):
    o_ref[...] = jnp.dot(x_ref[...], w_ref[...],
                         preferred_element_type=jnp.float32
                         ).astype(jnp.bfloat16)


def _project(x, w):
    d_in, d_out = w.shape
    return pl.pallas_call(
        _matmul_body,
        grid=(GRID,),
        in_specs=[
            pl.BlockSpec((BN, d_in), lambda i: (i, 0)),
            pl.BlockSpec((d_in, d_out), lambda i: (0, 0)),
        ],
        out_specs=pl.BlockSpec((BN, d_out), lambda i: (i, 0)),
        out_shape=jax.ShapeDtypeStruct((NP, d_out), jnp.bfloat16),
    )(x, w)


def _dense_body(a0_ref, a1_ref, h_ref, wr_ref, b_ref, wn_ref, hn_ref, pn_ref):
    i = pl.program_id(0)
    agg = a0_ref[...].astype(jnp.float32) + a1_ref[...].astype(jnp.float32)
    pre = agg + b_ref[...] + jnp.dot(h_ref[...], wr_ref[...],
                                     preferred_element_type=jnp.float32)
    hv = jnp.maximum(pre, 0.0)
    rows = i * BN + lax.broadcasted_iota(jnp.int32, (BN, FP), 0)
    hv = jnp.where(rows < N_REAL, hv, 0.0)
    hn_ref[...] = hv
    pn_ref[...] = jnp.dot(hv, wn_ref[...], preferred_element_type=jnp.float32
                          ).astype(jnp.bfloat16)


def _dense(a0, a1, h, w_root, b, w_next):
    d_in = h.shape[1]
    return pl.pallas_call(
        _dense_body,
        grid=(GRID,),
        in_specs=[
            pl.BlockSpec((BN, FP), lambda i: (i, 0)),
            pl.BlockSpec((BN, FP), lambda i: (i, 0)),
            pl.BlockSpec((BN, d_in), lambda i: (i, 0)),
            pl.BlockSpec((d_in, FP), lambda i: (0, 0)),
            pl.BlockSpec((1, FP), lambda i: (0, 0)),
            pl.BlockSpec((FP, FP), lambda i: (0, 0)),
        ],
        out_specs=[
            pl.BlockSpec((BN, FP), lambda i: (i, 0)),
            pl.BlockSpec((BN, FP), lambda i: (i, 0)),
        ],
        out_shape=[jax.ShapeDtypeStruct((NP, FP), jnp.float32),
                   jax.ShapeDtypeStruct((NP, FP), jnp.bfloat16)],
    )(a0, a1, h, w_root, b, w_next)


def _final_body(a0_ref, a1_ref, h_ref, wre_ref, wro_ref, bo_ref,
                wlin_ref, blin_ref, out_ref, acc_ref):
    i = pl.program_id(0)
    agg = a0_ref[...].astype(jnp.float32) + a1_ref[...].astype(jnp.float32)
    pre = (jnp.dot(agg, wre_ref[...], preferred_element_type=jnp.float32)
           + bo_ref[...]
           + jnp.dot(h_ref[...], wro_ref[...],
                     preferred_element_type=jnp.float32))
    hv = jnp.maximum(pre, 0.0)
    rows = i * BN + lax.broadcasted_iota(jnp.int32, (BN, 128), 0)
    hv = jnp.where(rows < N_REAL, hv, 0.0)
    s = jnp.sum(hv, axis=0, keepdims=True)

    @pl.when(i == 0)
    def _():
        acc_ref[...] = s

    @pl.when(i > 0)
    def _():
        acc_ref[...] = acc_ref[...] + s

    @pl.when(i == GRID - 1)
    def _():
        v = acc_ref[...]
        for t in range(4):
            v = jnp.maximum(
                jnp.dot(v, wlin_ref[t], preferred_element_type=jnp.float32)
                + blin_ref[t][None], 0.0)
        out_ref[...] = v


def _final(a0, a1, h, w_re, w_ro, b_o, w_lin, b_lin):
    return pl.pallas_call(
        _final_body,
        grid=(GRID,),
        in_specs=[
            pl.BlockSpec((BN, FP), lambda i: (i, 0)),
            pl.BlockSpec((BN, FP), lambda i: (i, 0)),
            pl.BlockSpec((BN, FP), lambda i: (i, 0)),
            pl.BlockSpec((FP, 128), lambda i: (0, 0)),
            pl.BlockSpec((FP, 128), lambda i: (0, 0)),
            pl.BlockSpec((1, 128), lambda i: (0, 0)),
            pl.BlockSpec((4, 128, 128), lambda i: (0, 0, 0)),
            pl.BlockSpec((4, 128), lambda i: (0, 0)),
        ],
        out_specs=pl.BlockSpec((1, 128), lambda i: (0, 0)),
        out_shape=jax.ShapeDtypeStruct((1, 128), jnp.float32),
        scratch_shapes=[pltpu.VMEM((1, 128), jnp.float32)],
    )(a0, a1, h, w_re, w_ro, b_o, w_lin, b_lin)


def _padw(w, r, c):
    out = jnp.zeros((r, c), jnp.float32)
    return out.at[:w.shape[0], :w.shape[1]].set(w)


def kernel(graph, n_feat, W_rel_in, b_in, W_root_in, W_rel_int, b_int,
           W_root_int, W_rel_out, b_out, W_root_out, W_lin, b_lin):
    f32 = jnp.float32
    x = jnp.zeros((NP, 128), f32).at[:N_REAL].set(n_feat)

    pad_e = EP - E_REAL
    pad_idx = jnp.full((pad_e,), NP - 1, jnp.int32)
    srcp = jnp.concatenate([graph[0], pad_idx]).reshape(NW, NOPS, CHUNK)
    dstp = jnp.concatenate([graph[1], pad_idx]).reshape(NW, NOPS, CHUNK)
    zeros = jnp.zeros((NP, FP), jnp.bfloat16)

    w_rel_in = _padw(W_rel_in, 128, FP)
    w_root_in = _padw(W_root_in, 128, FP)
    b0 = _padw(b_in[None], 1, FP)
    w_rel_mid = [_padw(W_rel_int[i], FP, FP) for i in range(W_rel_int.shape[0])]
    w_root_mid = [_padw(W_root_int[i], FP, FP) for i in range(W_root_int.shape[0])]
    b_mid = [_padw(b_int[i][None], 1, FP) for i in range(b_int.shape[0])]
    w_re_out = _padw(W_rel_out, FP, 128)
    w_ro_out = _padw(W_root_out, FP, 128)
    eye = jnp.eye(FP, dtype=f32)

    w_roots = [w_root_in] + w_root_mid
    bs = [b0] + b_mid
    w_nexts = w_rel_mid + [eye]

    p = _project(x, w_rel_in)
    h = x
    for l in range(len(w_roots)):
        agg = _sc_agg(p, srcp, dstp, zeros)
        h, p = _dense(agg[:NP], agg[NP:], h, w_roots[l], bs[l], w_nexts[l])

    agg = _sc_agg(p, srcp, dstp, zeros)
    v = _final(agg[:NP], agg[NP:], h, w_re_out, w_ro_out, b_out[None],
               W_lin, b_lin)
    return v[0]


# speedup vs baseline: 23.9066x; 1.0004x over previous
import functools

import jax
import jax.numpy as jnp
from jax import lax
from jax.experimental import pallas as pl
from jax.experimental.pallas import tpu as pltpu
from jax.experimental.pallas import tpu_sc as plsc

N_REAL = 10000
NP = 10240
FP = 32
NC, NS = 2, 16
NW = NC * NS
E_REAL = 320000
CHUNK = 640
EPW = 10240
NOPS = EPW // CHUNK
EP = EPW * NW
RPT = NP // NS
NBUF = 4
BN = 1024
GRID = NP // BN


def _sc_agg(p, srcp, dstp, zeros):
    mesh = plsc.VectorSubcoreMesh(core_axis_name="c", subcore_axis_name="s")

    @functools.partial(
        pl.kernel,
        out_type=jax.ShapeDtypeStruct((NC * NP, FP), jnp.bfloat16),
        mesh=mesh,
        compiler_params=pltpu.CompilerParams(use_tc_tiling_on_sc=False),
        scratch_types=[
            pltpu.VMEM((NOPS, CHUNK), jnp.int32),
            pltpu.VMEM((NOPS, CHUNK), jnp.int32),
            pltpu.VMEM((NBUF, CHUNK, FP), jnp.bfloat16),
            pltpu.VMEM((RPT, FP), jnp.bfloat16),
            pltpu.SemaphoreType.DMA((NBUF,)),
            pltpu.SemaphoreType.DMA((NBUF,)),
            pltpu.SemaphoreType.DMA((4,)),
            pltpu.VMEM_SHARED((NP, FP), jnp.bfloat16),
            pltpu.VMEM_SHARED((NP, FP), jnp.bfloat16),
        ],
    )
    def k(p_hbm, src_hbm, dst_hbm, z_hbm, agg_hbm,
          src_v, dst_v, msg_v, buf_v, semg, sems, semp, acc_sh, p_sh):
        cid = lax.axis_index("c")
        sid = lax.axis_index("s")
        wid = sid * NC + cid
        r0 = sid * RPT
        pro = [
            pltpu.make_async_copy(z_hbm.at[pl.ds(r0, RPT)],
                                  acc_sh.at[pl.ds(r0, RPT)], semp.at[0]),
            pltpu.make_async_copy(p_hbm.at[pl.ds(r0, RPT)],
                                  p_sh.at[pl.ds(r0, RPT)], semp.at[1]),
            pltpu.make_async_copy(src_hbm.at[wid], src_v, semp.at[2]),
            pltpu.make_async_copy(dst_hbm.at[wid], dst_v, semp.at[3]),
        ]
        for c in pro:
            c.start()
        for c in pro:
            c.wait()
        plsc.subcore_barrier()

        def fire_g(j, b):
            pltpu.make_async_copy(
                p_sh.at[src_v.at[j]], msg_v.at[b], semg.at[b]).start()

        def wait_g(b):
            pltpu.make_async_copy(
                p_sh.at[src_v.at[0]], msg_v.at[b], semg.at[b]).wait()

        def fire_s(j, b):
            pltpu.make_async_copy(
                msg_v.at[b], acc_sh.at[dst_v.at[j]],
                sems.at[b]).start(add=True)

        def wait_s(b):
            pltpu.make_async_copy(
                msg_v.at[b], acc_sh.at[dst_v.at[0]], sems.at[b]).wait()

        half = NBUF // 2
        for b in range(half):
            fire_g(b, b)

        def body(i, carry):
            j0 = i * NBUF
            for b in range(half):
                wait_g(b)
                fire_s(j0 + b, b)
            for b in range(half, NBUF):
                @pl.when(i > 0)
                def _():
                    wait_s(b)
                fire_g(j0 + b, b)
            for b in range(half, NBUF):
                wait_g(b)
                fire_s(j0 + b, b)
            for b in range(half):
                wait_s(b)
                jn = j0 + NBUF + b
                @pl.when(jn < NOPS)
                def _():
                    fire_g(jn, b)
            return carry

        lax.fori_loop(0, NOPS // NBUF, body, 0)
        for b in range(half, NBUF):
            wait_s(b)
        plsc.subcore_barrier()
        pltpu.sync_copy(acc_sh.at[pl.ds(r0, RPT)],
                        agg_hbm.at[pl.ds(cid * NP + r0, RPT)])

    return k(p, srcp, dstp, zeros)


def _matmul_body(x_ref, w_ref, o_ref):
    o_ref[...] = jnp.dot(x_ref[...], w_ref[...],
                         preferred_element_type=jnp.float32
                         ).astype(jnp.bfloat16)


def _project(x, w):
    d_in, d_out = w.shape
    return pl.pallas_call(
        _matmul_body,
        grid=(GRID,),
        in_specs=[
            pl.BlockSpec((BN, d_in), lambda i: (i, 0)),
            pl.BlockSpec((d_in, d_out), lambda i: (0, 0)),
        ],
        out_specs=pl.BlockSpec((BN, d_out), lambda i: (i, 0)),
        out_shape=jax.ShapeDtypeStruct((NP, d_out), jnp.bfloat16),
    )(x, w)


def _dense_body(a0_ref, a1_ref, h_ref, wr_ref, b_ref, wn_ref, hn_ref, pn_ref):
    i = pl.program_id(0)
    agg = a0_ref[...].astype(jnp.float32) + a1_ref[...].astype(jnp.float32)
    pre = agg + b_ref[...] + jnp.dot(h_ref[...], wr_ref[...],
                                     preferred_element_type=jnp.float32)
    hv = jnp.maximum(pre, 0.0)
    rows = i * BN + lax.broadcasted_iota(jnp.int32, (BN, FP), 0)
    hv = jnp.where(rows < N_REAL, hv, 0.0)
    hn_ref[...] = hv
    pn_ref[...] = jnp.dot(hv, wn_ref[...], preferred_element_type=jnp.float32
                          ).astype(jnp.bfloat16)


def _dense(a0, a1, h, w_root, b, w_next):
    d_in = h.shape[1]
    return pl.pallas_call(
        _dense_body,
        grid=(GRID,),
        in_specs=[
            pl.BlockSpec((BN, FP), lambda i: (i, 0)),
            pl.BlockSpec((BN, FP), lambda i: (i, 0)),
            pl.BlockSpec((BN, d_in), lambda i: (i, 0)),
            pl.BlockSpec((d_in, FP), lambda i: (0, 0)),
            pl.BlockSpec((1, FP), lambda i: (0, 0)),
            pl.BlockSpec((FP, FP), lambda i: (0, 0)),
        ],
        out_specs=[
            pl.BlockSpec((BN, FP), lambda i: (i, 0)),
            pl.BlockSpec((BN, FP), lambda i: (i, 0)),
        ],
        out_shape=[jax.ShapeDtypeStruct((NP, FP), jnp.float32),
                   jax.ShapeDtypeStruct((NP, FP), jnp.bfloat16)],
    )(a0, a1, h, w_root, b, w_next)


def _final_body(a0_ref, a1_ref, h_ref, wre_ref, wro_ref, bo_ref,
                wlin_ref, blin_ref, out_ref, acc_ref):
    i = pl.program_id(0)
    agg = a0_ref[...].astype(jnp.float32) + a1_ref[...].astype(jnp.float32)
    pre = (jnp.dot(agg, wre_ref[...], preferred_element_type=jnp.float32)
           + bo_ref[...]
           + jnp.dot(h_ref[...], wro_ref[...],
                     preferred_element_type=jnp.float32))
    hv = jnp.maximum(pre, 0.0)
    rows = i * BN + lax.broadcasted_iota(jnp.int32, (BN, 128), 0)
    hv = jnp.where(rows < N_REAL, hv, 0.0)
    s = jnp.sum(hv, axis=0, keepdims=True)

    @pl.when(i == 0)
    def _():
        acc_ref[...] = s

    @pl.when(i > 0)
    def _():
        acc_ref[...] = acc_ref[...] + s

    @pl.when(i == GRID - 1)
    def _():
        v = acc_ref[...]
        for t in range(4):
            v = jnp.maximum(
                jnp.dot(v, wlin_ref[t], preferred_element_type=jnp.float32)
                + blin_ref[t][None], 0.0)
        out_ref[...] = v


def _final(a0, a1, h, w_re, w_ro, b_o, w_lin, b_lin):
    return pl.pallas_call(
        _final_body,
        grid=(GRID,),
        in_specs=[
            pl.BlockSpec((BN, FP), lambda i: (i, 0)),
            pl.BlockSpec((BN, FP), lambda i: (i, 0)),
            pl.BlockSpec((BN, FP), lambda i: (i, 0)),
            pl.BlockSpec((FP, 128), lambda i: (0, 0)),
            pl.BlockSpec((FP, 128), lambda i: (0, 0)),
            pl.BlockSpec((1, 128), lambda i: (0, 0)),
            pl.BlockSpec((4, 128, 128), lambda i: (0, 0, 0)),
            pl.BlockSpec((4, 128), lambda i: (0, 0)),
        ],
        out_specs=pl.BlockSpec((1, 128), lambda i: (0, 0)),
        out_shape=jax.ShapeDtypeStruct((1, 128), jnp.float32),
        scratch_shapes=[pltpu.VMEM((1, 128), jnp.float32)],
    )(a0, a1, h, w_re, w_ro, b_o, w_lin, b_lin)


def _padw(w, r, c):
    out = jnp.zeros((r, c), jnp.float32)
    return out.at[:w.shape[0], :w.shape[1]].set(w)


def kernel(graph, n_feat, W_rel_in, b_in, W_root_in, W_rel_int, b_int,
           W_root_int, W_rel_out, b_out, W_root_out, W_lin, b_lin):
    f32 = jnp.float32
    x = jnp.zeros((NP, 128), f32).at[:N_REAL].set(n_feat)

    pad_e = EP - E_REAL
    pad_idx = jnp.full((pad_e,), NP - 1, jnp.int32)
    srcp = jnp.concatenate([graph[0], pad_idx]).reshape(NW, NOPS, CHUNK)
    dstp = jnp.concatenate([graph[1], pad_idx]).reshape(NW, NOPS, CHUNK)
    zeros = jnp.zeros((NP, FP), jnp.bfloat16)

    w_rel_in = _padw(W_rel_in, 128, FP)
    w_root_in = _padw(W_root_in, 128, FP)
    b0 = _padw(b_in[None], 1, FP)
    w_rel_mid = [_padw(W_rel_int[i], FP, FP) for i in range(W_rel_int.shape[0])]
    w_root_mid = [_padw(W_root_int[i], FP, FP) for i in range(W_root_int.shape[0])]
    b_mid = [_padw(b_int[i][None], 1, FP) for i in range(b_int.shape[0])]
    w_re_out = _padw(W_rel_out, FP, 128)
    w_ro_out = _padw(W_root_out, FP, 128)
    eye = jnp.eye(FP, dtype=f32)

    w_roots = [w_root_in] + w_root_mid
    bs = [b0] + b_mid
    w_nexts = w_rel_mid + [eye]

    p = _project(x, w_rel_in)
    h = x
    for l in range(len(w_roots)):
        agg = _sc_agg(p, srcp, dstp, zeros)
        h, p = _dense(agg[:NP], agg[NP:], h, w_roots[l], bs[l], w_nexts[l])

    agg = _sc_agg(p, srcp, dstp, zeros)
    v = _final(agg[:NP], agg[NP:], h, w_re_out, w_ro_out, b_out[None],
               W_lin, b_lin)
    return v[0]
